# baseline (speedup 1.0000x reference)
import functools

import jax
import jax.numpy as jnp
from jax import lax
from jax.experimental import pallas as pl
from jax.experimental.pallas import tpu as pltpu
from jax.experimental.pallas import tpu_sc as plsc

_B = 16384
_J = 12
_V = 160
_D = 8
_NC = 2
_NS = 16
_NW = _NC * _NS
_BPW = _B // _NW
_RB = _BPW // 16


@functools.partial(
    pl.kernel,
    out_type=jax.ShapeDtypeStruct((_B * 3,), jnp.float32),
    mesh=plsc.VectorSubcoreMesh(core_axis_name="c", subcore_axis_name="s"),
    compiler_params=pltpu.CompilerParams(needs_layout_passes=False),
    scratch_types=[
        pltpu.VMEM((_V, _D), jnp.float32),
        pltpu.VMEM((3, _J * _D), jnp.float32),
        pltpu.VMEM((48,), jnp.float32),
        pltpu.VMEM((_J * _V,), jnp.float32),
        pltpu.VMEM((_J * _V,), jnp.float32),
        pltpu.VMEM((_J * _V,), jnp.float32),
        pltpu.VMEM((_BPW * _J,), jnp.int32),
        pltpu.VMEM((_BPW * 3,), jnp.float32),
    ],
)
def _sc_classify(emb_hbm, w3_hbm, b3_hbm, idx_hbm, out_hbm,
                 emb_v, w3_v, b3_v, l0_v, l1_v, l2_v, idx_v, out_v):
    wid = lax.axis_index("s") * _NC + lax.axis_index("c")
    base = wid * _BPW
    pltpu.sync_copy(idx_hbm.at[pl.ds(base * _J, _BPW * _J)], idx_v)
    pltpu.sync_copy(emb_hbm, emb_v)
    pltpu.sync_copy(w3_hbm, w3_v)
    pltpu.sync_copy(b3_hbm, b3_v)

    lanes = jax.lax.iota(jnp.int32, 16)

    def build_j(j, carry):
        wcols = [jnp.broadcast_to(j * _D + d, (16,)).astype(jnp.int32)
                 for d in range(_D)]
        wvs = [[plsc.load_gather(w3_v, [jnp.full((16,), c, jnp.int32), wcols[d]])
                for d in range(_D)] for c in range(3)]
        for chunk in range(_V // 16):
            vidx = lanes + chunk * 16
            evs = []
            for d in range(_D):
                ev = plsc.load_gather(emb_v, [vidx, jnp.full((16,), d, jnp.int32)])
                evs.append(jnp.maximum(ev, 0.0))
            for c, lcv in enumerate((l0_v, l1_v, l2_v)):
                acc = evs[0] * wvs[c][0]
                for d in range(1, _D):
                    acc = acc + evs[d] * wvs[c][d]
                lcv[pl.ds(j * _V + chunk * 16, 16)] = acc
        return carry

    lax.fori_loop(0, _J, build_j, 0)

    b3v = [b3_v[pl.ds(c * 16, 16)] for c in range(3)]
    lanes_j = lanes * _J
    lanes_3 = lanes * 3

    def body(rb, carry):
        pbase = rb * (16 * _J)
        acc0, acc1, acc2 = b3v
        for j in range(_J):
            pidx = lanes_j + (pbase + j)
            vj = plsc.load_gather(idx_v, [pidx])
            fidx = vj + (j * _V)
            acc0 = acc0 + plsc.load_gather(l0_v, [fidx])
            acc1 = acc1 + plsc.load_gather(l1_v, [fidx])
            acc2 = acc2 + plsc.load_gather(l2_v, [fidx])
        obase = rb * 48
        plsc.store_scatter(out_v, [lanes_3 + obase], acc0)
        plsc.store_scatter(out_v, [lanes_3 + (obase + 1)], acc1)
        plsc.store_scatter(out_v, [lanes_3 + (obase + 2)], acc2)
        return carry

    lax.fori_loop(0, _RB, body, 0)
    pltpu.sync_copy(out_v, out_hbm.at[pl.ds(base * 3, _BPW * 3)])


def kernel(pose_indices, image, emb_table, W3, b3):
    del image
    out_flat = _sc_classify(emb_table, W3, jnp.repeat(b3, 16),
                            pose_indices.astype(jnp.int32).reshape(-1))
    return out_flat.reshape(_B, 3)


# speedup vs baseline: 1.4004x; 1.4004x over previous
import functools

import jax
import jax.numpy as jnp
from jax import lax
from jax.experimental import pallas as pl
from jax.experimental.pallas import tpu as pltpu
from jax.experimental.pallas import tpu_sc as plsc

_B = 16384
_J = 12
_V = 160
_D = 8
_NC = 2
_NS = 16
_NW = _NC * _NS
_BPW = _B // _NW
_RB = _BPW // 16


def _table_body(emb_ref, w3_ref, b3_ref, l_ref):
    e = jnp.maximum(emb_ref[...], 0.0)
    w = w3_ref[...]
    for j in range(_J):
        blk = w[:, _D * j:_D * (j + 1)]
        lj = lax.dot_general(blk, e, (((1,), (1,)), ((), ())),
                             preferred_element_type=jnp.float32)
        if j == 0:
            lj = lj + b3_ref[...]
        l_ref[:, _V * j:_V * (j + 1)] = lj


_build_table = pl.pallas_call(
    _table_body,
    out_shape=jax.ShapeDtypeStruct((3, _J * _V), jnp.float32),
)


@functools.partial(
    pl.kernel,
    out_type=jax.ShapeDtypeStruct((_B, 3), jnp.float32),
    mesh=plsc.VectorSubcoreMesh(core_axis_name="c", subcore_axis_name="s"),
    compiler_params=pltpu.CompilerParams(needs_layout_passes=False),
    scratch_types=[
        pltpu.VMEM((3, _J * _V), jnp.float32),
        pltpu.VMEM((_BPW // 2, _J), jnp.int32),
        pltpu.VMEM((_BPW // 2, 3), jnp.float32),
    ],
)
def _sc_lookup(l_hbm, idx_hbm, out_hbm, l_v, idx_v, out_v):
    wid = lax.axis_index("s") * _NC + lax.axis_index("c")
    base = wid * _BPW
    half = _BPW // 2
    pltpu.sync_copy(l_hbm, l_v)

    lanes = jax.lax.iota(jnp.int32, 16)
    csplat = [jnp.full((16,), c, jnp.int32) for c in range(3)]

    def body(rb, carry):
        rows = lanes + rb * 16
        acc0 = jnp.zeros((16,), jnp.float32)
        acc1 = jnp.zeros((16,), jnp.float32)
        acc2 = jnp.zeros((16,), jnp.float32)
        for j in range(_J):
            vj = plsc.load_gather(idx_v, [rows, jnp.full((16,), j, jnp.int32)])
            fidx = vj + (j * _V)
            acc0 = acc0 + plsc.load_gather(l_v, [csplat[0], fidx])
            acc1 = acc1 + plsc.load_gather(l_v, [csplat[1], fidx])
            acc2 = acc2 + plsc.load_gather(l_v, [csplat[2], fidx])
        for c, acc in enumerate((acc0, acc1, acc2)):
            plsc.store_scatter(out_v, [rows, jnp.full((16,), c, jnp.int32)], acc)
        return carry

    for h in range(2):
        hbase = base + h * half
        pltpu.sync_copy(idx_hbm.at[pl.ds(hbase, half), :], idx_v)
        lax.fori_loop(0, half // 16, body, 0)
        pltpu.sync_copy(out_v, out_hbm.at[pl.ds(hbase, half), :])


def kernel(pose_indices, image, emb_table, W3, b3):
    del image
    l_table = _build_table(emb_table, W3, b3.reshape(3, 1))
    return _sc_lookup(l_table, pose_indices)
